# Initial kernel scaffold; baseline (speedup 1.0000x reference)
import jax
import jax.numpy as jnp
from jax.experimental import pallas as pl


def kernel(ufeats, ifeats, Pw, Pb, Ww, Wb, edge_index):
    raise NotImplementedError("write your pallas kernel here")


# speedup vs baseline: 1.2602x; 1.2602x over previous
import functools

import jax
import jax.numpy as jnp
from jax import lax
from jax.experimental import pallas as pl
from jax.experimental.pallas import tpu as pltpu
from jax.experimental.pallas import tpu_sc as plsc

_N_USERS = 10000
_N_ITEMS = 10000
_E = 320000
_D = 128
_NB = 2
_NC = 5

_NCORES = 2
_NSUB = 16
_NW = _NCORES * _NSUB
_EPW = _E // _NW
_C = 128
_NFULL = _EPW // _C
_TAIL = _EPW - _NFULL * _C


def _fu_body(u_ref, w_ref, b_ref, o_ref):
    o_ref[...] = (
        jnp.dot(u_ref[...], w_ref[...], preferred_element_type=jnp.float32,
                precision=lax.Precision.HIGHEST)
        + b_ref[0:1, :]
    )


def _compute_fu(ufeats, wcat, btile):
    rows = 1000
    grid = _N_USERS // rows
    return pl.pallas_call(
        _fu_body,
        grid=(grid,),
        in_specs=[
            pl.BlockSpec((rows, _D), lambda i: (i, 0)),
            pl.BlockSpec((_D, _NB * _D), lambda i: (0, 0)),
            pl.BlockSpec((8, _NB * _D), lambda i: (0, 0)),
        ],
        out_specs=pl.BlockSpec((rows, _NB * _D), lambda i: (i, 0)),
        out_shape=jax.ShapeDtypeStruct((_N_USERS, _NB * _D), jnp.float32),
    )(ufeats, wcat, btile)


def _sc_body(fu_hbm, if_hbm, src_hbm, dst_hbm, wtab_hbm, out_hbm,
             idxs_v, idxd_v, idxs_t, idxd_t, hi_v, fu_v, out_v, wtab_v,
             sem0, sem1):
    cid = lax.axis_index("c")
    sid = lax.axis_index("s")
    wid = sid * _NCORES + cid
    base = wid * _EPW

    pltpu.sync_copy(wtab_hbm, wtab_v)
    w0 = [wtab_v[c] for c in range(_NC)]
    w1 = [wtab_v[_NC + c] for c in range(_NC)]
    wb = [wtab_v[2 * _NC + c] for c in range(_NC)]
    iota = lax.iota(jnp.int32, 16)
    zf = jnp.zeros((16,), jnp.float32)

    def do_chunk(ebase, n, idxs_ref, idxd_ref):
        pltpu.sync_copy(src_hbm.at[pl.ds(ebase, n)], idxs_ref)
        pltpu.sync_copy(dst_hbm.at[pl.ds(ebase, n)], idxd_ref)
        g1 = pltpu.async_copy(if_hbm.at[idxs_ref], hi_v.at[pl.ds(0, n)], sem0)
        g2 = pltpu.async_copy(fu_hbm.at[idxd_ref], fu_v.at[pl.ds(0, n)], sem1)
        g1.wait()
        g2.wait()
        for g in range(n // 16):
            rows = g * 16 + iota

            def dstep(i, carry, rows=rows):
                a0, a1 = carry
                for k in range(4):
                    dd = i * 4 + k
                    col = jnp.full((16,), 0, jnp.int32) + dd
                    hi = plsc.load_gather(hi_v, [rows, col])
                    f0 = plsc.load_gather(fu_v, [rows, col])
                    f1 = plsc.load_gather(fu_v, [rows, col + _D])
                    a0 = a0 + hi * f0
                    a1 = a1 + hi * f1
                return a0, a1

            s0, s1 = lax.fori_loop(0, _D // 4, dstep, (zf, zf))
            for c in range(_NC):
                val = s0 * w0[c] + s1 * w1[c] + wb[c]
                ccol = jnp.full((16,), c, jnp.int32)
                plsc.store_scatter(out_v, [rows, ccol], val)
        pltpu.sync_copy(out_v.at[pl.ds(0, n)], out_hbm.at[pl.ds(ebase, n)])

    def chunk_body(j, carry):
        do_chunk(base + j * _C, _C, idxs_v, idxd_v)
        return carry

    lax.fori_loop(0, _NFULL, chunk_body, 0)
    if _TAIL:
        do_chunk(base + _NFULL * _C, _TAIL, idxs_t, idxd_t)


_sc_kernel = functools.partial(
    pl.kernel,
    out_type=jax.ShapeDtypeStruct((_E, _NC), jnp.float32),
    mesh=plsc.VectorSubcoreMesh(core_axis_name="c", subcore_axis_name="s"),
    compiler_params=pltpu.CompilerParams(needs_layout_passes=False),
    scratch_types=[
        pltpu.VMEM((_C,), jnp.int32),
        pltpu.VMEM((_C,), jnp.int32),
        pltpu.VMEM((_TAIL,), jnp.int32),
        pltpu.VMEM((_TAIL,), jnp.int32),
        pltpu.VMEM((_C, _D), jnp.float32),
        pltpu.VMEM((_C, _NB * _D), jnp.float32),
        pltpu.VMEM((_C, _NC), jnp.float32),
        pltpu.VMEM((16, 16), jnp.float32),
        pltpu.SemaphoreType.DMA,
        pltpu.SemaphoreType.DMA,
    ],
)(_sc_body)


def kernel(ufeats, ifeats, Pw, Pb, Ww, Wb, edge_index):
    wcat = jnp.concatenate([Pw[0].T, Pw[1].T], axis=1)
    bcat = jnp.concatenate([Pb[0], Pb[1]], axis=0)
    btile = jnp.tile(bcat[None, :], (8, 1))
    src = edge_index[0].astype(jnp.int32)
    dst = edge_index[1].astype(jnp.int32)
    wtab = jnp.zeros((16, 16), jnp.float32)
    wtab = wtab.at[0:_NC, :].set(Ww[:, 0:1])
    wtab = wtab.at[_NC:2 * _NC, :].set(Ww[:, 1:2])
    wtab = wtab.at[2 * _NC:3 * _NC, :].set(Wb[:, None])

    fu = _compute_fu(ufeats, wcat, btile)
    return _sc_kernel(fu, ifeats, src, dst, wtab)


# speedup vs baseline: 1.4644x; 1.1620x over previous
import functools

import jax
import jax.numpy as jnp
from jax import lax
from jax.experimental import pallas as pl
from jax.experimental.pallas import tpu as pltpu
from jax.experimental.pallas import tpu_sc as plsc

_N_USERS = 10000
_N_ITEMS = 10000
_E = 320000
_D = 128
_NB = 2
_NC = 5

_NCORES = 2
_NSUB = 16
_NW = _NCORES * _NSUB
_EPW = _E // _NW
_C = 128
_SBC = 16
_SB = _SBC * _C
_NSB = _EPW // _SB
_REMC = (_EPW - _NSB * _SB) // _C
_TAIL = _EPW - _NSB * _SB - _REMC * _C


def _fu_body(u_ref, w_ref, b_ref, o_ref):
    o_ref[...] = (
        jnp.dot(u_ref[...], w_ref[...], preferred_element_type=jnp.float32,
                precision=lax.Precision.HIGHEST)
        + b_ref[0:1, :]
    )


def _compute_fu(ufeats, wcat, btile):
    rows = 1000
    grid = _N_USERS // rows
    return pl.pallas_call(
        _fu_body,
        grid=(grid,),
        in_specs=[
            pl.BlockSpec((rows, _D), lambda i: (i, 0)),
            pl.BlockSpec((_D, _NB * _D), lambda i: (0, 0)),
            pl.BlockSpec((8, _NB * _D), lambda i: (0, 0)),
        ],
        out_specs=pl.BlockSpec((rows, _NB * _D), lambda i: (i, 0)),
        out_shape=jax.ShapeDtypeStruct((_N_USERS, _NB * _D), jnp.float32),
    )(ufeats, wcat, btile)


def _sc_body(fu_hbm, if_hbm, src_hbm, dst_hbm, wtab_hbm, out_hbm,
             idxs_v, idxd_v, hi0, hi1, fu0, fu1, out_v, wtab_v,
             semh0, semh1, semf0, semf1):
    cid = lax.axis_index("c")
    sid = lax.axis_index("s")
    wid = sid * _NCORES + cid
    base = wid * _EPW

    pltpu.sync_copy(wtab_hbm, wtab_v)
    w0 = [wtab_v[c] for c in range(_NC)]
    w1 = [wtab_v[_NC + c] for c in range(_NC)]
    wb = [wtab_v[2 * _NC + c] for c in range(_NC)]
    iota = lax.iota(jnp.int32, 16)
    zf = jnp.zeros((16,), jnp.float32)
    zi = jnp.zeros((16,), jnp.int32)

    his = (hi0, hi1)
    fus = (fu0, fu1)
    semh = (semh0, semh1)
    semf = (semf0, semf1)

    def gather_chunk(c_local, b):
        isl = idxs_v.at[pl.ds(c_local * _C, _C)]
        dsl = idxd_v.at[pl.ds(c_local * _C, _C)]
        pltpu.async_copy(if_hbm.at[isl], his[b], semh[b])
        pltpu.async_copy(fu_hbm.at[dsl], fus[b], semf[b])

    def wait_chunk(c_local, b):
        isl = idxs_v.at[pl.ds(c_local * _C, _C)]
        dsl = idxd_v.at[pl.ds(c_local * _C, _C)]
        pltpu.make_async_copy(if_hbm.at[isl], his[b], semh[b]).wait()
        pltpu.make_async_copy(fu_hbm.at[dsl], fus[b], semf[b]).wait()

    def compute(orow_base, b, ngroups):
        hi_v = his[b]
        fu_v = fus[b]

        def gbody(g, _):
            rows = g * 16 + iota

            def dstep(i, carry):
                a0, a0b, a1, a1b, col = carry
                for k in range(8):
                    ck = col + k if k else col
                    hi = plsc.load_gather(hi_v, [rows, ck])
                    f0 = plsc.load_gather(fu_v, [rows, ck])
                    f1 = plsc.load_gather(fu_v, [rows, ck + _D])
                    if k % 2 == 0:
                        a0 = a0 + hi * f0
                        a1 = a1 + hi * f1
                    else:
                        a0b = a0b + hi * f0
                        a1b = a1b + hi * f1
                return a0, a0b, a1, a1b, col + 8

            a0, a0b, a1, a1b, _unused = lax.fori_loop(
                0, _D // 8, dstep, (zf, zf, zf, zf, zi))
            s0 = a0 + a0b
            s1 = a1 + a1b
            orow5 = (orow_base + g * 16 + iota) * _NC
            for c in range(_NC):
                val = s0 * w0[c] + s1 * w1[c] + wb[c]
                plsc.store_scatter(out_v, [orow5 + c], val)
            return 0

        lax.fori_loop(0, ngroups, gbody, 0)

    def run_block(sb_base, nchunks, stage_n):
        pltpu.sync_copy(src_hbm.at[pl.ds(sb_base, stage_n)],
                        idxs_v.at[pl.ds(0, stage_n)])
        pltpu.sync_copy(dst_hbm.at[pl.ds(sb_base, stage_n)],
                        idxd_v.at[pl.ds(0, stage_n)])
        gather_chunk(0, 0)
        npairs = nchunks // 2

        def pbody(p, _):
            c0 = 2 * p
            c1 = c0 + 1
            gather_chunk(c1, 1)
            wait_chunk(c0, 0)
            compute(c0 * _C, 0, _C // 16)

            @pl.when(p < npairs - 1)
            def _prefetch():
                gather_chunk(c0 + 2, 0)

            wait_chunk(c1, 1)
            compute(c1 * _C, 1, _C // 16)
            return 0

        lax.fori_loop(0, npairs, pbody, 0)

    def sb_body(t, _):
        sb_base = base + t * _SB
        run_block(sb_base, _SBC, _SB)
        pltpu.sync_copy(out_v, out_hbm.at[pl.ds(sb_base * _NC, _SB * _NC)])
        return 0

    lax.fori_loop(0, _NSB, sb_body, 0)

    rem_base = base + _NSB * _SB
    rem_edges = _REMC * _C + _TAIL
    run_block(rem_base, _REMC, rem_edges)
    isl = idxs_v.at[pl.ds(_REMC * _C, _TAIL)]
    dsl = idxd_v.at[pl.ds(_REMC * _C, _TAIL)]
    g1 = pltpu.async_copy(if_hbm.at[isl], hi0.at[pl.ds(0, _TAIL)], semh0)
    g2 = pltpu.async_copy(fu_hbm.at[dsl], fu0.at[pl.ds(0, _TAIL)], semf0)
    g1.wait()
    g2.wait()
    compute(_REMC * _C, 0, _TAIL // 16)
    pltpu.sync_copy(out_v.at[pl.ds(0, rem_edges * _NC)],
                    out_hbm.at[pl.ds(rem_base * _NC, rem_edges * _NC)])


_sc_kernel = functools.partial(
    pl.kernel,
    out_type=jax.ShapeDtypeStruct((_E * _NC,), jnp.float32),
    mesh=plsc.VectorSubcoreMesh(core_axis_name="c", subcore_axis_name="s"),
    compiler_params=pltpu.CompilerParams(needs_layout_passes=False),
    scratch_types=[
        pltpu.VMEM((_SB,), jnp.int32),
        pltpu.VMEM((_SB,), jnp.int32),
        pltpu.VMEM((_C, _D), jnp.float32),
        pltpu.VMEM((_C, _D), jnp.float32),
        pltpu.VMEM((_C, _NB * _D), jnp.float32),
        pltpu.VMEM((_C, _NB * _D), jnp.float32),
        pltpu.VMEM((_SB * _NC,), jnp.float32),
        pltpu.VMEM((16, 16), jnp.float32),
        pltpu.SemaphoreType.DMA,
        pltpu.SemaphoreType.DMA,
        pltpu.SemaphoreType.DMA,
        pltpu.SemaphoreType.DMA,
    ],
)(_sc_body)


def kernel(ufeats, ifeats, Pw, Pb, Ww, Wb, edge_index):
    wcat = jnp.concatenate([Pw[0].T, Pw[1].T], axis=1)
    bcat = jnp.concatenate([Pb[0], Pb[1]], axis=0)
    btile = jnp.tile(bcat[None, :], (8, 1))
    src = edge_index[0].astype(jnp.int32)
    dst = edge_index[1].astype(jnp.int32)
    wtab = jnp.zeros((16, 16), jnp.float32)
    wtab = wtab.at[0:_NC, :].set(Ww[:, 0:1])
    wtab = wtab.at[_NC:2 * _NC, :].set(Ww[:, 1:2])
    wtab = wtab.at[2 * _NC:3 * _NC, :].set(Wb[:, None])

    fu = _compute_fu(ufeats, wcat, btile)
    return _sc_kernel(fu, ifeats, src, dst, wtab).reshape(_E, _NC)
